# Initial kernel scaffold; baseline (speedup 1.0000x reference)
import jax
import jax.numpy as jnp
from jax.experimental import pallas as pl


def kernel(g2m_efeat, grid_nfeat, mesh_nfeat, We, Ws, Wd, be1, We2, be2, ge, bge, Ws1, bs1, Ws2, bs2, gs, bgs, Wd1, bd1, Wd2, bd2, gd, bgd, src_idx, dst_idx):
    raise NotImplementedError("write your pallas kernel here")


# speedup vs baseline: 2.5356x; 2.5356x over previous
import functools

import jax
import jax.numpy as jnp
from jax import lax
from jax.experimental import pallas as pl
from jax.experimental.pallas import tpu as pltpu
from jax.experimental.pallas import tpu_sc as plsc

_E = 320000
_NG = 50000
_NM = 10000
_D = 128

_B = 512
_W = 128
_NB = _E // _B

_GBLK = 1000
_C = 80


def _ln(y, g, b):
    m = jnp.mean(y, axis=-1, keepdims=True)
    c = y - m
    v = jnp.mean(c * c, axis=-1, keepdims=True)
    return c * lax.rsqrt(v + 1e-5) * g + b


def _grid_body(x_ref, ws_ref, ws1_ref, bs1_ref, ws2_ref, bs2_ref, gs_ref,
               bgs_ref, out_ref, proj_ref):
    x = x_ref[...]
    proj_ref[...] = jnp.dot(x, ws_ref[...], preferred_element_type=jnp.float32)
    hs = jax.nn.silu(
        jnp.dot(x, ws1_ref[...], preferred_element_type=jnp.float32)
        + bs1_ref[...])
    y = jnp.dot(hs, ws2_ref[...], preferred_element_type=jnp.float32) + bs2_ref[...]
    out_ref[...] = x + _ln(y, gs_ref[...], bgs_ref[...])


def _grid_branch(grid_nfeat, Ws, Ws1, bs1, Ws2, bs2, gs, bgs):
    n = grid_nfeat.shape[0]
    row = lambda i: (i, 0)
    full = lambda i: (0, 0)
    wspec = pl.BlockSpec((_D, _D), full)
    vspec = pl.BlockSpec((1, _D), full)
    return pl.pallas_call(
        _grid_body,
        grid=(n // _GBLK,),
        in_specs=[pl.BlockSpec((_GBLK, _D), row), wspec, wspec, vspec, wspec,
                  vspec, vspec, vspec],
        out_specs=[pl.BlockSpec((_GBLK, _D), row)] * 2,
        out_shape=[jax.ShapeDtypeStruct((n, _D), jnp.float32)] * 2,
        compiler_params=pltpu.CompilerParams(
            dimension_semantics=("parallel",)),
    )(grid_nfeat, Ws, Ws1, bs1.reshape(1, _D), Ws2, bs2.reshape(1, _D),
      gs.reshape(1, _D), bgs.reshape(1, _D))


def _mesh_pre_body(x_ref, wd_ref, be1_ref, out_ref):
    out_ref[...] = (jnp.dot(x_ref[...], wd_ref[...],
                            preferred_element_type=jnp.float32) + be1_ref[...])


def _mesh_pre(mesh_nfeat, Wd, be1):
    n = mesh_nfeat.shape[0]
    return pl.pallas_call(
        _mesh_pre_body,
        grid=(n // _GBLK,),
        in_specs=[pl.BlockSpec((_GBLK, _D), lambda i: (i, 0)),
                  pl.BlockSpec((_D, _D), lambda i: (0, 0)),
                  pl.BlockSpec((1, _D), lambda i: (0, 0))],
        out_specs=pl.BlockSpec((_GBLK, _D), lambda i: (i, 0)),
        out_shape=jax.ShapeDtypeStruct((n, _D), jnp.float32),
        compiler_params=pltpu.CompilerParams(
            dimension_semantics=("parallel",)),
    )(mesh_nfeat, Wd, be1.reshape(1, _D))


def _sc_gather_body(per_w, n_chunks, table_hbm, idx_hbm, out_hbm, idx_v,
                    rows_v, sem):
    wid = lax.axis_index("s") * 2 + lax.axis_index("c")
    base = wid * per_w

    def body(c, carry):
        off = base + c * _C
        pltpu.sync_copy(idx_hbm.at[pl.ds(off, _C)], idx_v)
        pltpu.async_copy(table_hbm.at[idx_v], rows_v, sem).wait()
        pltpu.sync_copy(rows_v, out_hbm.at[pl.ds(off, _C)])
        return carry

    lax.fori_loop(0, n_chunks, body, 0)


def _sc_gather(table, idx):
    n = idx.shape[0]
    info = plsc.get_sparse_core_info()
    nw = info.num_cores * info.num_subcores
    per_w = n // nw
    n_chunks = per_w // _C
    mesh = plsc.VectorSubcoreMesh(core_axis_name="c", subcore_axis_name="s")
    kern = functools.partial(
        pl.kernel,
        mesh=mesh,
        out_type=jax.ShapeDtypeStruct((n, _D), jnp.float32),
        scratch_types=[
            pltpu.VMEM((_C,), jnp.int32),
            pltpu.VMEM((_C, _D), jnp.float32),
            pltpu.SemaphoreType.DMA,
        ],
    )(functools.partial(_sc_gather_body, per_w, n_chunks))
    return kern(table, idx)


def _edge_body(e_ref, gsrc_ref, drow_ref, dcol_ref, mesh_ref, we_ref,
               we2_ref, be2_ref, ge_ref, bge_ref, out_ref, acc_ref):
    i = pl.program_id(0)

    @pl.when(i == 0)
    def _():
        acc_ref[...] = jnp.zeros((_NM + _W, _D), jnp.float32)

    drow = drow_ref[0]
    dcol = dcol_ref[0]
    d0 = jnp.min(drow)
    dmax = jnp.max(drow)
    base = (d0 // 8) * 8
    nwin = (dmax - base) // _W + 1

    iota_bw = lax.broadcasted_iota(jnp.int32, (_B, _W), 1)
    iota_wb = lax.broadcasted_iota(jnp.int32, (_W, _B), 0)

    def gwin(w, md):
        wb = base + w * _W
        oh = (iota_bw == (dcol - wb)).astype(jnp.float32)
        mwin = mesh_ref[pl.ds(wb, _W), :]
        return md + jnp.dot(oh, mwin, preferred_element_type=jnp.float32)

    mdst = lax.fori_loop(0, nwin, gwin, jnp.zeros((_B, _D), jnp.float32))

    h = jax.nn.silu(
        jnp.dot(e_ref[...], we_ref[...], preferred_element_type=jnp.float32)
        + gsrc_ref[...] + mdst)
    y = jnp.dot(h, we2_ref[...], preferred_element_type=jnp.float32) + be2_ref[...]
    mlp = _ln(y, ge_ref[...], bge_ref[...])

    def swin(w, carry):
        wb = base + w * _W
        oht = (iota_wb == (drow - wb)).astype(jnp.float32)
        acc_ref[pl.ds(wb, _W), :] += jnp.dot(
            oht, mlp, preferred_element_type=jnp.float32)
        return carry

    lax.fori_loop(0, nwin, swin, 0)

    @pl.when(i == _NB - 1)
    def _():
        out_ref[...] = acc_ref[pl.ds(0, _NM), :]


def _edge_agg(g2m_efeat, gsrc, dst_idx, mesh_proj_pad, We, We2, be2, ge, bge):
    drow = dst_idx.reshape(_NB, 1, _B)
    dcol = dst_idx.reshape(_NB, _B, 1)
    row = lambda i: (i, 0)
    full = lambda i: (0, 0)
    return pl.pallas_call(
        _edge_body,
        grid=(_NB,),
        in_specs=[
            pl.BlockSpec((_B, _D), row),
            pl.BlockSpec((_B, _D), row),
            pl.BlockSpec((1, 1, _B), lambda i: (i, 0, 0)),
            pl.BlockSpec((1, _B, 1), lambda i: (i, 0, 0)),
            pl.BlockSpec((_NM + _W, _D), full),
            pl.BlockSpec((_D, _D), full),
            pl.BlockSpec((_D, _D), full),
            pl.BlockSpec((1, _D), full),
            pl.BlockSpec((1, _D), full),
            pl.BlockSpec((1, _D), full),
        ],
        out_specs=pl.BlockSpec((_NM, _D), full),
        out_shape=jax.ShapeDtypeStruct((_NM, _D), jnp.float32),
        scratch_shapes=[pltpu.VMEM((_NM + _W, _D), jnp.float32)],
        compiler_params=pltpu.CompilerParams(
            dimension_semantics=("arbitrary",)),
    )(g2m_efeat, gsrc, drow, dcol, mesh_proj_pad, We, We2,
      be2.reshape(1, _D), ge.reshape(1, _D), bge.reshape(1, _D))


def _mesh_post_body(agg_ref, x_ref, wd1a_ref, wd1b_ref, bd1_ref, wd2_ref,
                    bd2_ref, gd_ref, bgd_ref, out_ref):
    x = x_ref[...]
    hd = jax.nn.silu(
        jnp.dot(agg_ref[...], wd1a_ref[...], preferred_element_type=jnp.float32)
        + jnp.dot(x, wd1b_ref[...], preferred_element_type=jnp.float32)
        + bd1_ref[...])
    y = jnp.dot(hd, wd2_ref[...], preferred_element_type=jnp.float32) + bd2_ref[...]
    out_ref[...] = x + _ln(y, gd_ref[...], bgd_ref[...])


def _mesh_post(agg, mesh_nfeat, Wd1, bd1, Wd2, bd2, gd, bgd):
    row = lambda i: (i, 0)
    full = lambda i: (0, 0)
    wspec = pl.BlockSpec((_D, _D), full)
    vspec = pl.BlockSpec((1, _D), full)
    return pl.pallas_call(
        _mesh_post_body,
        grid=(_NM // _GBLK,),
        in_specs=[pl.BlockSpec((_GBLK, _D), row),
                  pl.BlockSpec((_GBLK, _D), row),
                  wspec, wspec, vspec, wspec, vspec, vspec, vspec],
        out_specs=pl.BlockSpec((_GBLK, _D), row),
        out_shape=jax.ShapeDtypeStruct((_NM, _D), jnp.float32),
        compiler_params=pltpu.CompilerParams(
            dimension_semantics=("parallel",)),
    )(agg, mesh_nfeat, Wd1[:_D], Wd1[_D:], bd1.reshape(1, _D), Wd2,
      bd2.reshape(1, _D), gd.reshape(1, _D), bgd.reshape(1, _D))


def kernel(g2m_efeat, grid_nfeat, mesh_nfeat, We, Ws, Wd, be1, We2, be2, ge,
           bge, Ws1, bs1, Ws2, bs2, gs, bgs, Wd1, bd1, Wd2, bd2, gd, bgd,
           src_idx, dst_idx):
    grid_out, grid_proj = _grid_branch(grid_nfeat, Ws, Ws1, bs1, Ws2, bs2,
                                       gs, bgs)
    mesh_proj = _mesh_pre(mesh_nfeat, Wd, be1)
    mesh_proj_pad = jnp.pad(mesh_proj, ((0, _W), (0, 0)))
    gsrc = _sc_gather(grid_proj, src_idx)
    agg = _edge_agg(g2m_efeat, gsrc, dst_idx, mesh_proj_pad, We, We2, be2,
                    ge, bge)
    mesh_out = _mesh_post(agg, mesh_nfeat, Wd1, bd1, Wd2, bd2, gd, bgd)
    return (grid_out, mesh_out)
